# baseline (speedup 1.0000x reference)
import functools

import jax
import jax.numpy as jnp
from jax import lax
from jax.experimental import pallas as pl
from jax.experimental.pallas import tpu as pltpu
from jax.experimental.pallas import tpu_sc as plsc

N = 10000
E = 160000
D = 128
NPAD = 10240
NC = 2
NS = 16
CH = 128
HC = CH // 2
EPAD = 163840
CPW = EPAD // (NC * NS * CH)
NACC = NPAD
ROWS_PER_TILE = NACC // NS
BUFR = 32
BN = 1024
F32 = jnp.float32
I32 = jnp.int32

_mesh = plsc.VectorSubcoreMesh(core_axis_name="c", subcore_axis_name="s")


def _zero_buf(buf, nrow):
    @pl.loop(0, nrow)
    def _(i):
        for v in range(D // 16):
            buf[i, pl.ds(v * 16, 16)] = jnp.zeros((16,), F32)


def _sc_body(with_deg, h, s0, d0, s1, d1, s2, d2, *rest):
    if with_deg:
        (agg_out, deg_out, sidx, didx, r0, r1, buf, zdeg, ones1,
         acc, deg_sp, ma0, mb0, sa0, sb0) = rest
    else:
        (agg_out, sidx, didx, r0, r1, buf, acc, ma0, mb0, sa0, sb0) = rest
    rows = (r0, r1)
    sems = (ma0, mb0)
    ssems = (sa0, sb0)

    def start_gather(slot, k):
        pltpu.async_copy(h.at[sidx.at[k]], rows[slot], sems[slot])

    def wait_gather(slot, k):
        pltpu.make_async_copy(h.at[sidx.at[k]], rows[slot],
                              sems[slot]).wait()
    c = lax.axis_index("c")
    s = lax.axis_index("s")
    wid = c * NS + s
    b0 = s * ROWS_PER_TILE
    b0d = s * (NPAD // NS)
    cb = wid * CPW

    if with_deg:
        for v in range(D // 16):
            zdeg[pl.ds(v * 16, 16)] = jnp.zeros((16,), F32)
        for v in range(CH // 16):
            ones1[pl.ds(v * 16, 16)] = jnp.ones((16,), F32)

    for r, (srcs, dsts) in enumerate(((s0, d0), (s1, d1), (s2, d2))):
        pltpu.sync_copy(srcs.at[pl.ds(cb, CPW)], sidx)
        pltpu.sync_copy(dsts.at[pl.ds(cb, CPW)], didx)

        _zero_buf(buf, BUFR)

        @pl.loop(0, ROWS_PER_TILE // BUFR)
        def _(k):
            pltpu.async_copy(buf, acc.at[pl.ds(b0 + k * BUFR, BUFR)],
                             sems[0])

        @pl.loop(0, ROWS_PER_TILE // BUFR)
        def _(k):
            pltpu.make_async_copy(buf, acc.at[pl.ds(b0, BUFR)],
                                  sems[0]).wait()

        if with_deg:
            @pl.loop(0, NPAD // NS // D)
            def _(k):
                pltpu.sync_copy(zdeg, deg_sp.at[pl.ds(b0d + k * D, D)])

        start_gather(0, 0)
        start_gather(1, 1)
        plsc.subcore_barrier()

        @pl.loop(0, CPW, step=2)
        def _(j):
            wait_gather(0, j)
            pltpu.async_copy(rows[0], acc.at[didx.at[j]], ssems[0],
                             add=True)
            if with_deg:
                pltpu.sync_copy(ones1, deg_sp.at[didx.at[j]], add=True)

            wait_gather(1, j + 1)
            pltpu.async_copy(rows[1], acc.at[didx.at[j + 1]], ssems[1],
                             add=True)
            if with_deg:
                pltpu.sync_copy(ones1, deg_sp.at[didx.at[j + 1]], add=True)

            pltpu.make_async_copy(rows[0], acc.at[didx.at[j]],
                                  ssems[0]).wait()

            @pl.when(j + 2 < CPW)
            def _():
                start_gather(0, j + 2)

            pltpu.make_async_copy(rows[1], acc.at[didx.at[j + 1]],
                                  ssems[1]).wait()

            @pl.when(j + 3 < CPW)
            def _():
                start_gather(1, j + 3)

        plsc.subcore_barrier()

        pltpu.sync_copy(acc.at[pl.ds(b0, ROWS_PER_TILE)],
                        agg_out.at[c, r, pl.ds(b0, ROWS_PER_TILE)])
        if with_deg:
            pltpu.sync_copy(
                deg_sp.at[pl.ds(b0d, NPAD // NS)],
                deg_out.at[pl.ds((c * 3 + r) * NPAD + b0d, NPAD // NS)])


def _make_sc_kernel(with_deg):
    out_type = [jax.ShapeDtypeStruct((NC, 3, NPAD, D), F32)]
    scratch = [
        pltpu.VMEM((CPW, CH), I32),
        pltpu.VMEM((CPW, CH), I32),
        pltpu.VMEM((CH, D), F32),
        pltpu.VMEM((CH, D), F32),
        pltpu.VMEM((BUFR, D), F32),
    ]
    if with_deg:
        out_type.append(jax.ShapeDtypeStruct((NC * 3 * NPAD,), F32))
        scratch += [
            pltpu.VMEM((D,), F32),
            pltpu.VMEM((CH,), F32),
        ]
    scratch += [
        pltpu.VMEM_SHARED((NACC, D), F32),
    ]
    if with_deg:
        scratch.append(pltpu.VMEM_SHARED((NPAD,), F32))
    scratch += [pltpu.SemaphoreType.DMA] * 4
    return pl.kernel(
        functools.partial(_sc_body, with_deg),
        out_type=out_type,
        mesh=_mesh,
        scratch_types=scratch,
    )


_sc_agg_deg = _make_sc_kernel(True)
_sc_agg = _make_sc_kernel(False)


def _tc_body(with_skip, *refs):
    if with_skip:
        agg_ref, deg_ref, w_ref, b_ref, h_ref, wsl_ref, bsl_ref, o_ref = refs
    else:
        agg_ref, deg_ref, w_ref, b_ref, o_ref = refs
    a = agg_ref[...]
    dg = deg_ref[...]
    acc = jnp.zeros((BN, D), F32)
    for r in range(3):
        inv = 1.0 / jnp.clip(dg[0, r] + dg[1, r], 1.0, None)
        ar = (a[0, r] + a[1, r]) * inv
        acc = acc + jnp.dot(ar, w_ref[r], preferred_element_type=F32) + b_ref[r]
    if with_skip:
        acc = acc + jnp.dot(h_ref[...], wsl_ref[...],
                            preferred_element_type=F32) + bsl_ref[...]
        o_ref[...] = acc
    else:
        o_ref[...] = jnp.maximum(acc, 0.0)


def _make_tc_kernel(with_skip):
    in_specs = [
        pl.BlockSpec((NC, 3, BN, D), lambda i: (0, 0, i, 0)),
        pl.BlockSpec((NC, 3, BN, 1), lambda i: (0, 0, i, 0)),
        pl.BlockSpec((3, D, D), lambda i: (0, 0, 0)),
        pl.BlockSpec((3, 1, D), lambda i: (0, 0, 0)),
    ]
    if with_skip:
        in_specs += [
            pl.BlockSpec((BN, D), lambda i: (i, 0)),
            pl.BlockSpec((D, D), lambda i: (0, 0)),
            pl.BlockSpec((1, D), lambda i: (0, 0)),
        ]
    return pl.pallas_call(
        functools.partial(_tc_body, with_skip),
        grid=(NPAD // BN,),
        in_specs=in_specs,
        out_specs=pl.BlockSpec((BN, D), lambda i: (i, 0)),
        out_shape=jax.ShapeDtypeStruct((NPAD, D), F32),
    )


_tc_layer = _make_tc_kernel(False)
_tc_layer_skip = _make_tc_kernel(True)


def _pad_edges(ei):
    npad = EPAD - E
    pad_src = jnp.arange(npad, dtype=I32) % N
    pad_dst = N + (jnp.arange(npad, dtype=I32) % (NACC - N))
    s = jnp.concatenate([ei[0], pad_src]).reshape(EPAD // CH, CH)
    d = jnp.concatenate([ei[1], pad_dst]).reshape(EPAD // CH, CH)
    return s, d


def kernel(x, edge_index_r0, edge_index_r1, edge_index_r2,
           W0_0, b0_0, W0_1, b0_1, W0_2, b0_2,
           W1_0, b1_0, W1_1, b1_1, W1_2, b1_2,
           W2_0, b2_0, W2_1, b2_1, W2_2, b2_2,
           W_sl, b_sl):
    s0, d0 = _pad_edges(edge_index_r0)
    s1, d1 = _pad_edges(edge_index_r1)
    s2, d2 = _pad_edges(edge_index_r2)
    Ws = [jnp.stack([W0_0, W0_1, W0_2]),
          jnp.stack([W1_0, W1_1, W1_2]),
          jnp.stack([W2_0, W2_1, W2_2])]
    bs = [jnp.stack([b0_0, b0_1, b0_2]).reshape(3, 1, D),
          jnp.stack([b1_0, b1_1, b1_2]).reshape(3, 1, D),
          jnp.stack([b2_0, b2_1, b2_2]).reshape(3, 1, D)]

    agg, deg = _sc_agg_deg(x, s0, d0, s1, d1, s2, d2)
    deg4 = deg.reshape(NC, 3, NPAD, 1)
    h = _tc_layer(agg, deg4, Ws[0], bs[0])
    (agg,) = _sc_agg(h, s0, d0, s1, d1, s2, d2)
    h2 = _tc_layer(agg, deg4, Ws[1], bs[1])
    (agg,) = _sc_agg(h2, s0, d0, s1, d1, s2, d2)
    out = _tc_layer_skip(agg, deg4, Ws[2], bs[2], h2, W_sl,
                         b_sl.reshape(1, D))
    return out[:N]


# speedup vs baseline: 1.2512x; 1.2512x over previous
import functools

import jax
import jax.numpy as jnp
from jax import lax
from jax.experimental import pallas as pl
from jax.experimental.pallas import tpu as pltpu
from jax.experimental.pallas import tpu_sc as plsc

N = 10000
E = 160000
D = 128
NPAD = 10240
NC = 2
NS = 16
CH = 128
HC = CH // 2
EPAD = 163840
CPW = EPAD // (NC * NS * CH)
NACC = NPAD
ROWS_PER_TILE = NACC // NS
BUFR = 32
BN = 1024
F32 = jnp.float32
I32 = jnp.int32

_mesh = plsc.VectorSubcoreMesh(core_axis_name="c", subcore_axis_name="s")


def _zero_buf(buf, nrow):
    @pl.loop(0, nrow)
    def _(i):
        for v in range(D // 16):
            buf[i, pl.ds(v * 16, 16)] = jnp.zeros((16,), F32)


def _sc_body(with_deg, h, s0, d0, s1, d1, s2, d2, *rest):
    if with_deg:
        (agg_out, deg_out, sidx, didx, r0, r1, buf, zdeg, ones1,
         acc, deg_sp, ma0, mb0) = rest
    else:
        (agg_out, sidx, didx, r0, r1, buf, acc, ma0, mb0) = rest
    rows = (r0, r1)
    sems = (ma0, mb0)

    def start_gather(slot, k):
        pltpu.async_copy(h.at[sidx.at[k]], rows[slot], sems[slot])

    def wait_gather(slot, k):
        pltpu.make_async_copy(h.at[sidx.at[k]], rows[slot],
                              sems[slot]).wait()
    c = lax.axis_index("c")
    s = lax.axis_index("s")
    wid = c * NS + s
    b0 = s * ROWS_PER_TILE
    b0d = s * (NPAD // NS)
    cb = wid * CPW

    if with_deg:
        for v in range(D // 16):
            zdeg[pl.ds(v * 16, 16)] = jnp.zeros((16,), F32)
        for v in range(CH // 16):
            ones1[pl.ds(v * 16, 16)] = jnp.ones((16,), F32)

    for r, (srcs, dsts) in enumerate(((s0, d0), (s1, d1), (s2, d2))):
        pltpu.sync_copy(srcs.at[pl.ds(cb, CPW)], sidx)
        pltpu.sync_copy(dsts.at[pl.ds(cb, CPW)], didx)

        _zero_buf(buf, BUFR)

        @pl.loop(0, ROWS_PER_TILE // BUFR)
        def _(k):
            pltpu.async_copy(buf, acc.at[pl.ds(b0 + k * BUFR, BUFR)],
                             sems[0])

        @pl.loop(0, ROWS_PER_TILE // BUFR)
        def _(k):
            pltpu.make_async_copy(buf, acc.at[pl.ds(b0, BUFR)],
                                  sems[0]).wait()

        if with_deg:
            @pl.loop(0, NPAD // NS // D)
            def _(k):
                pltpu.sync_copy(zdeg, deg_sp.at[pl.ds(b0d + k * D, D)])

        start_gather(0, 0)
        start_gather(1, 1)
        plsc.subcore_barrier()

        @pl.loop(0, CPW, step=2)
        def _(j):
            wait_gather(0, j)
            pltpu.sync_copy(rows[0], acc.at[didx.at[j]], add=True)
            if with_deg:
                pltpu.sync_copy(ones1, deg_sp.at[didx.at[j]], add=True)

            @pl.when(j + 2 < CPW)
            def _():
                start_gather(0, j + 2)

            wait_gather(1, j + 1)
            pltpu.sync_copy(rows[1], acc.at[didx.at[j + 1]], add=True)
            if with_deg:
                pltpu.sync_copy(ones1, deg_sp.at[didx.at[j + 1]], add=True)

            @pl.when(j + 3 < CPW)
            def _():
                start_gather(1, j + 3)

        plsc.subcore_barrier()

        pltpu.sync_copy(acc.at[pl.ds(b0, ROWS_PER_TILE)],
                        agg_out.at[c, r, pl.ds(b0, ROWS_PER_TILE)])
        if with_deg:
            pltpu.sync_copy(
                deg_sp.at[pl.ds(b0d, NPAD // NS)],
                deg_out.at[pl.ds((c * 3 + r) * NPAD + b0d, NPAD // NS)])


def _make_sc_kernel(with_deg):
    out_type = [jax.ShapeDtypeStruct((NC, 3, NPAD, D), F32)]
    scratch = [
        pltpu.VMEM((CPW, CH), I32),
        pltpu.VMEM((CPW, CH), I32),
        pltpu.VMEM((CH, D), F32),
        pltpu.VMEM((CH, D), F32),
        pltpu.VMEM((BUFR, D), F32),
    ]
    if with_deg:
        out_type.append(jax.ShapeDtypeStruct((NC * 3 * NPAD,), F32))
        scratch += [
            pltpu.VMEM((D,), F32),
            pltpu.VMEM((CH,), F32),
        ]
    scratch += [
        pltpu.VMEM_SHARED((NACC, D), F32),
    ]
    if with_deg:
        scratch.append(pltpu.VMEM_SHARED((NPAD,), F32))
    scratch += [pltpu.SemaphoreType.DMA] * 2
    return pl.kernel(
        functools.partial(_sc_body, with_deg),
        out_type=out_type,
        mesh=_mesh,
        scratch_types=scratch,
    )


_sc_agg_deg = _make_sc_kernel(True)
_sc_agg = _make_sc_kernel(False)


def _tc_body(with_skip, *refs):
    if with_skip:
        agg_ref, deg_ref, w_ref, b_ref, h_ref, wsl_ref, bsl_ref, o_ref = refs
    else:
        agg_ref, deg_ref, w_ref, b_ref, o_ref = refs
    a = agg_ref[...]
    dg = deg_ref[...]
    acc = jnp.zeros((BN, D), F32)
    for r in range(3):
        inv = 1.0 / jnp.clip(dg[0, r] + dg[1, r], 1.0, None)
        ar = (a[0, r] + a[1, r]) * inv
        acc = acc + jnp.dot(ar, w_ref[r], preferred_element_type=F32) + b_ref[r]
    if with_skip:
        acc = acc + jnp.dot(h_ref[...], wsl_ref[...],
                            preferred_element_type=F32) + bsl_ref[...]
        o_ref[...] = acc
    else:
        o_ref[...] = jnp.maximum(acc, 0.0)


def _make_tc_kernel(with_skip):
    in_specs = [
        pl.BlockSpec((NC, 3, BN, D), lambda i: (0, 0, i, 0)),
        pl.BlockSpec((NC, 3, BN, 1), lambda i: (0, 0, i, 0)),
        pl.BlockSpec((3, D, D), lambda i: (0, 0, 0)),
        pl.BlockSpec((3, 1, D), lambda i: (0, 0, 0)),
    ]
    if with_skip:
        in_specs += [
            pl.BlockSpec((BN, D), lambda i: (i, 0)),
            pl.BlockSpec((D, D), lambda i: (0, 0)),
            pl.BlockSpec((1, D), lambda i: (0, 0)),
        ]
    return pl.pallas_call(
        functools.partial(_tc_body, with_skip),
        grid=(NPAD // BN,),
        in_specs=in_specs,
        out_specs=pl.BlockSpec((BN, D), lambda i: (i, 0)),
        out_shape=jax.ShapeDtypeStruct((NPAD, D), F32),
    )


_tc_layer = _make_tc_kernel(False)
_tc_layer_skip = _make_tc_kernel(True)


def _pad_edges(ei):
    npad = EPAD - E
    pad_src = jnp.arange(npad, dtype=I32) % N
    pad_dst = N + (jnp.arange(npad, dtype=I32) % (NACC - N))
    s = jnp.concatenate([ei[0], pad_src]).reshape(EPAD // CH, CH)
    d = jnp.concatenate([ei[1], pad_dst]).reshape(EPAD // CH, CH)
    return s, d


def kernel(x, edge_index_r0, edge_index_r1, edge_index_r2,
           W0_0, b0_0, W0_1, b0_1, W0_2, b0_2,
           W1_0, b1_0, W1_1, b1_1, W1_2, b1_2,
           W2_0, b2_0, W2_1, b2_1, W2_2, b2_2,
           W_sl, b_sl):
    s0, d0 = _pad_edges(edge_index_r0)
    s1, d1 = _pad_edges(edge_index_r1)
    s2, d2 = _pad_edges(edge_index_r2)
    Ws = [jnp.stack([W0_0, W0_1, W0_2]),
          jnp.stack([W1_0, W1_1, W1_2]),
          jnp.stack([W2_0, W2_1, W2_2])]
    bs = [jnp.stack([b0_0, b0_1, b0_2]).reshape(3, 1, D),
          jnp.stack([b1_0, b1_1, b1_2]).reshape(3, 1, D),
          jnp.stack([b2_0, b2_1, b2_2]).reshape(3, 1, D)]

    agg, deg = _sc_agg_deg(x, s0, d0, s1, d1, s2, d2)
    deg4 = deg.reshape(NC, 3, NPAD, 1)
    h = _tc_layer(agg, deg4, Ws[0], bs[0])
    (agg,) = _sc_agg(h, s0, d0, s1, d1, s2, d2)
    h2 = _tc_layer(agg, deg4, Ws[1], bs[1])
    (agg,) = _sc_agg(h2, s0, d0, s1, d1, s2, d2)
    out = _tc_layer_skip(agg, deg4, Ws[2], bs[2], h2, W_sl,
                         b_sl.reshape(1, D))
    return out[:N]


# speedup vs baseline: 1.2587x; 1.0060x over previous
import functools

import jax
import jax.numpy as jnp
from jax import lax
from jax.experimental import pallas as pl
from jax.experimental.pallas import tpu as pltpu
from jax.experimental.pallas import tpu_sc as plsc

N = 10000
E = 160000
D = 128
NPAD = 10240
NC = 2
NS = 16
CH = 128
EPAD = 163840
CPW = EPAD // (NC * NS * CH)
NACC = NPAD
ROWS_PER_TILE = NACC // NS
BUFR = 32
BN = 1024
F32 = jnp.float32
I32 = jnp.int32

_mesh = plsc.VectorSubcoreMesh(core_axis_name="c", subcore_axis_name="s")


def _zero_buf(buf, nrow):
    @pl.loop(0, nrow)
    def _(i):
        for v in range(D // 16):
            buf[i, pl.ds(v * 16, 16)] = jnp.zeros((16,), F32)


def _sc_body(with_deg, h, s0, d0, s1, d1, s2, d2, *rest):
    if with_deg:
        (agg_out, deg_out, sidx, didx, r0, r1, buf, zdeg, ones1,
         acc, deg_sp, ma0, mb0) = rest
    else:
        (agg_out, sidx, didx, r0, r1, buf, acc, ma0, mb0) = rest
    rows = (r0, r1)
    sems = (ma0, mb0)

    def start_gather(slot, k):
        pltpu.async_copy(h.at[sidx.at[k]], rows[slot], sems[slot])

    def wait_gather(slot, k):
        pltpu.make_async_copy(h.at[sidx.at[k]], rows[slot],
                              sems[slot]).wait()
    c = lax.axis_index("c")
    s = lax.axis_index("s")
    wid = c * NS + s
    b0 = s * ROWS_PER_TILE
    b0d = s * (NPAD // NS)
    cb = wid * CPW

    if with_deg:
        for v in range(D // 16):
            zdeg[pl.ds(v * 16, 16)] = jnp.zeros((16,), F32)
        for v in range(CH // 16):
            ones1[pl.ds(v * 16, 16)] = jnp.ones((16,), F32)

    for r, (srcs, dsts) in enumerate(((s0, d0), (s1, d1), (s2, d2))):
        pltpu.sync_copy(srcs.at[pl.ds(cb, CPW)], sidx)
        pltpu.sync_copy(dsts.at[pl.ds(cb, CPW)], didx)

        _zero_buf(buf, BUFR)

        @pl.loop(0, ROWS_PER_TILE // BUFR)
        def _(k):
            pltpu.async_copy(buf, acc.at[pl.ds(b0 + k * BUFR, BUFR)],
                             sems[0])

        @pl.loop(0, ROWS_PER_TILE // BUFR)
        def _(k):
            pltpu.make_async_copy(buf, acc.at[pl.ds(b0, BUFR)],
                                  sems[0]).wait()

        if with_deg:
            @pl.loop(0, NPAD // NS // D)
            def _(k):
                pltpu.sync_copy(zdeg, deg_sp.at[pl.ds(b0d + k * D, D)])

        start_gather(0, 0)
        start_gather(1, 1)
        plsc.subcore_barrier()

        @pl.loop(0, CPW, step=2)
        def _(j):
            wait_gather(0, j)
            pltpu.sync_copy(rows[0], acc.at[didx.at[j]], add=True)
            if with_deg:
                pltpu.sync_copy(ones1, deg_sp.at[didx.at[j]], add=True)

            @pl.when(j + 2 < CPW)
            def _():
                start_gather(0, j + 2)

            wait_gather(1, j + 1)
            pltpu.sync_copy(rows[1], acc.at[didx.at[j + 1]], add=True)
            if with_deg:
                pltpu.sync_copy(ones1, deg_sp.at[didx.at[j + 1]], add=True)

            @pl.when(j + 3 < CPW)
            def _():
                start_gather(1, j + 3)

        plsc.subcore_barrier()

        pltpu.sync_copy(acc.at[pl.ds(b0, ROWS_PER_TILE)],
                        agg_out.at[c, r, pl.ds(b0, ROWS_PER_TILE)])
        if with_deg:
            pltpu.sync_copy(
                deg_sp.at[pl.ds(b0d, NPAD // NS)],
                deg_out.at[pl.ds((c * 3 + r) * NPAD + b0d, NPAD // NS)])


def _make_sc_kernel(with_deg):
    out_type = [jax.ShapeDtypeStruct((NC, 3, NPAD, D), F32)]
    scratch = [
        pltpu.VMEM((CPW, CH), I32),
        pltpu.VMEM((CPW, CH), I32),
        pltpu.VMEM((CH, D), F32),
        pltpu.VMEM((CH, D), F32),
        pltpu.VMEM((BUFR, D), F32),
    ]
    if with_deg:
        out_type.append(jax.ShapeDtypeStruct((NC * 3 * NPAD,), F32))
        scratch += [
            pltpu.VMEM((D,), F32),
            pltpu.VMEM((CH,), F32),
        ]
    scratch += [
        pltpu.VMEM_SHARED((NACC, D), F32),
    ]
    if with_deg:
        scratch.append(pltpu.VMEM_SHARED((NPAD,), F32))
    scratch += [pltpu.SemaphoreType.DMA] * 2
    return pl.kernel(
        functools.partial(_sc_body, with_deg),
        out_type=out_type,
        mesh=_mesh,
        scratch_types=scratch,
    )


_sc_agg_deg = _make_sc_kernel(True)
_sc_agg = _make_sc_kernel(False)


def _tc_body(with_skip, bn, *refs):
    if with_skip:
        agg_ref, deg_ref, w_ref, b_ref, h_ref, wsl_ref, bsl_ref, o_ref = refs
    else:
        agg_ref, deg_ref, w_ref, b_ref, o_ref = refs
    a = agg_ref[...]
    dg = deg_ref[...]
    acc = jnp.zeros((bn, D), F32)
    for r in range(3):
        inv = 1.0 / jnp.clip(dg[0, r] + dg[1, r], 1.0, None)
        ar = (a[0, r] + a[1, r]) * inv
        acc = acc + jnp.dot(ar, w_ref[r], preferred_element_type=F32) + b_ref[r]
    if with_skip:
        acc = acc + jnp.dot(h_ref[...], wsl_ref[...],
                            preferred_element_type=F32) + bsl_ref[...]
        o_ref[...] = acc
    else:
        o_ref[...] = jnp.maximum(acc, 0.0)


def _make_tc_kernel(with_skip):
    bn, nout = (1000, N) if with_skip else (BN, NPAD)
    in_specs = [
        pl.BlockSpec((NC, 3, bn, D), lambda i: (0, 0, i, 0)),
        pl.BlockSpec((NC, 3, bn, 1), lambda i: (0, 0, i, 0)),
        pl.BlockSpec((3, D, D), lambda i: (0, 0, 0)),
        pl.BlockSpec((3, 1, D), lambda i: (0, 0, 0)),
    ]
    if with_skip:
        in_specs += [
            pl.BlockSpec((bn, D), lambda i: (i, 0)),
            pl.BlockSpec((D, D), lambda i: (0, 0)),
            pl.BlockSpec((1, D), lambda i: (0, 0)),
        ]
    return pl.pallas_call(
        functools.partial(_tc_body, with_skip, bn),
        grid=(nout // bn,),
        in_specs=in_specs,
        out_specs=pl.BlockSpec((bn, D), lambda i: (i, 0)),
        out_shape=jax.ShapeDtypeStruct((nout, D), F32),
    )


_tc_layer = _make_tc_kernel(False)
_tc_layer_skip = _make_tc_kernel(True)


def _pad_edges(ei):
    npad = EPAD - E
    pad_src = jnp.arange(npad, dtype=I32) % N
    pad_dst = N + (jnp.arange(npad, dtype=I32) % (NACC - N))
    s = jnp.concatenate([ei[0], pad_src]).reshape(EPAD // CH, CH)
    d = jnp.concatenate([ei[1], pad_dst]).reshape(EPAD // CH, CH)
    return s, d


def kernel(x, edge_index_r0, edge_index_r1, edge_index_r2,
           W0_0, b0_0, W0_1, b0_1, W0_2, b0_2,
           W1_0, b1_0, W1_1, b1_1, W1_2, b1_2,
           W2_0, b2_0, W2_1, b2_1, W2_2, b2_2,
           W_sl, b_sl):
    s0, d0 = _pad_edges(edge_index_r0)
    s1, d1 = _pad_edges(edge_index_r1)
    s2, d2 = _pad_edges(edge_index_r2)
    Ws = [jnp.stack([W0_0, W0_1, W0_2]),
          jnp.stack([W1_0, W1_1, W1_2]),
          jnp.stack([W2_0, W2_1, W2_2])]
    bs = [jnp.stack([b0_0, b0_1, b0_2]).reshape(3, 1, D),
          jnp.stack([b1_0, b1_1, b1_2]).reshape(3, 1, D),
          jnp.stack([b2_0, b2_1, b2_2]).reshape(3, 1, D)]

    agg, deg = _sc_agg_deg(x, s0, d0, s1, d1, s2, d2)
    deg4 = deg.reshape(NC, 3, NPAD, 1)
    h = _tc_layer(agg, deg4, Ws[0], bs[0])
    (agg,) = _sc_agg(h, s0, d0, s1, d1, s2, d2)
    h2 = _tc_layer(agg, deg4, Ws[1], bs[1])
    (agg,) = _sc_agg(h2, s0, d0, s1, d1, s2, d2)
    return _tc_layer_skip(agg, deg4, Ws[2], bs[2], h2, W_sl,
                          b_sl.reshape(1, D))
